# baseline (speedup 1.0000x reference)
import functools

import jax
import jax.numpy as jnp
from jax import lax
from jax.experimental import pallas as pl
from jax.experimental.pallas import tpu as pltpu
from jax.experimental.pallas import tpu_sc as plsc

D_MODEL = 64
SCALE = 8.0

NC = 2
NS = 16
NW = NC * NS

LANE = 128
CHUNK_G = 2
NBUF = 4


def _build(n_groups, n_groups_w, n_chunks):
    mesh = plsc.VectorSubcoreMesh(core_axis_name="c", subcore_axis_name="s")

    @functools.partial(
        pl.kernel,
        out_type=jax.ShapeDtypeStruct((n_groups, LANE, D_MODEL), jnp.float32),
        mesh=mesh,
        compiler_params=pltpu.CompilerParams(use_tc_tiling_on_sc=False),
        scratch_types=[
            pltpu.VMEM((n_groups_w, LANE), jnp.int32),
            pltpu.VMEM((NBUF, CHUNK_G, LANE, D_MODEL), jnp.float32),
            pltpu.SemaphoreType.DMA((NBUF,)),
            pltpu.SemaphoreType.DMA((NBUF,)),
        ],
    )
    def emb(x_hbm, tab_hbm, out_hbm, idx_v, rows_v, gsem, osem):
        wid = lax.axis_index("s") * NC + lax.axis_index("c")
        gbase = wid * n_groups_w
        pltpu.sync_copy(x_hbm.at[pl.ds(gbase, n_groups_w)], idx_v)

        def fire_gather(g, b):
            for j in range(CHUNK_G):
                pltpu.async_copy(
                    tab_hbm.at[idx_v.at[g * CHUNK_G + j]],
                    rows_v.at[b, j],
                    gsem.at[b],
                )

        def drain_gather(b):
            for j in range(CHUNK_G):
                pltpu.make_async_copy(
                    tab_hbm.at[idx_v.at[j]], rows_v.at[b, j], gsem.at[b]
                ).wait()

        def fire_out(g, b):
            pltpu.async_copy(
                rows_v.at[b],
                out_hbm.at[pl.ds(gbase + g * CHUNK_G, CHUNK_G)],
                osem.at[b],
            )

        def wait_out(b):
            pltpu.make_async_copy(
                rows_v.at[b], out_hbm.at[pl.ds(0, CHUNK_G)], osem.at[b]
            ).wait()

        for g in range(NBUF - 1):
            fire_gather(g, g)

        def chunk_iter(t, carry):
            for b in range(NBUF):
                g = t * NBUF + b
                drain_gather(b)

                def scale_row(r, c):
                    for j in range(CHUNK_G):
                        for k in range(D_MODEL // 16):
                            sl = pl.ds(16 * k, 16)
                            rows_v[b, j, r, sl] = rows_v[b, j, r, sl] * SCALE
                    return c

                lax.fori_loop(0, LANE, scale_row, 0)
                fire_out(g, b)
                nb = (b + NBUF - 1) % NBUF

                @pl.when(g + NBUF - 1 < n_chunks)
                def _prep():
                    @pl.when(g >= 1)
                    def _w():
                        wait_out(nb)

                    fire_gather(g + NBUF - 1, nb)

            return carry

        lax.fori_loop(0, n_chunks // NBUF, chunk_iter, 0)
        for b in range(NBUF):
            wait_out(b)

    return emb


def kernel(x, table):
    b0, s = x.shape
    n = b0 * s
    assert n % (LANE * NW) == 0
    n_groups = n // LANE
    n_groups_w = n_groups // NW
    n_chunks = n_groups_w // CHUNK_G
    assert n_chunks % NBUF == 0
    xg = x.reshape(n_groups, LANE).astype(jnp.int32)
    out = _build(n_groups, n_groups_w, n_chunks)(xg, table)
    return out.reshape(b0, s, D_MODEL)
